# Initial kernel scaffold; baseline (speedup 1.0000x reference)
import jax
import jax.numpy as jnp
from jax.experimental import pallas as pl


def kernel(obs, Lambda0, C, beta, sigma):
    raise NotImplementedError("write your pallas kernel here")


# speedup vs baseline: 4.1772x; 4.1772x over previous
import functools
import math

import jax
import jax.numpy as jnp
from jax import lax
from jax.experimental import pallas as pl
from jax.experimental.pallas import tpu as pltpu
from jax.experimental.pallas import tpu_sc as plsc

_T = 512
_P = 64
_KPT = 32
_N = _T * _P
_EPS = 1e-5
_AREA = 1.0

_NW = 32
_DPW = _T // _NW
_HD = _DPW + _KPT
_HE = _HD * _P
_OE = _DPW * _P
_NQ = _P // 16


def _sc_kers_body(xp_hbm, yp_hbm, ws_hbm, out_hbm, xv, yv, wv, ov):
    cid = lax.axis_index("c")
    sid = lax.axis_index("s")
    w = cid * 16 + sid
    base = w * _OE
    pltpu.sync_copy(xp_hbm.at[pl.ds(base, _HE)], xv)
    pltpu.sync_copy(yp_hbm.at[pl.ds(base, _HE)], yv)
    pltpu.sync_copy(ws_hbm, wv)
    d0base = w * _DPW

    def day_body(dd, carry):
        cb = (_KPT + dd) * _P
        cx = [xv[pl.ds(cb + 16 * q, 16)] for q in range(_NQ)]
        cy = [yv[pl.ds(cb + 16 * q, 16)] for q in range(_NQ)]
        kmax = jnp.minimum(jnp.int32(_KPT), d0base + dd)

        def k_body(k, accs):
            hb = (_KPT + dd - k) * _P
            wkv = wv[pl.ds((k - 1) * 16, 16)]

            def j_body(jc, accs2):
                a = list(accs2)
                xc = xv[pl.ds(hb + jc * 16, 16)]
                yc = yv[pl.ds(hb + jc * 16, 16)]
                for jj in range(16):
                    xjv = jnp.full((16,), xc[jj], jnp.float32)
                    yjv = jnp.full((16,), yc[jj], jnp.float32)
                    for q in range(_NQ):
                        dx = cx[q] - xjv
                        dy = cy[q] - yjv
                        e = jnp.exp(-(dx * dx + dy * dy))
                        a[q] = a[q] + wkv * e
                return tuple(a)

            return lax.fori_loop(0, _P // 16, j_body, accs)

        z = jnp.zeros((16,), jnp.float32)
        accs = lax.fori_loop(1, kmax + 1, k_body, (z, z, z, z))
        ob = dd * _P
        for q in range(_NQ):
            ov[pl.ds(ob + 16 * q, 16)] = accs[q]
        return carry

    lax.fori_loop(0, _DPW, day_body, 0)
    pltpu.sync_copy(ov, out_hbm.at[pl.ds(base, _OE)])


_sc_kers = functools.partial(
    pl.kernel,
    out_type=jax.ShapeDtypeStruct((_N,), jnp.float32),
    mesh=plsc.VectorSubcoreMesh(
        core_axis_name="c", subcore_axis_name="s", num_cores=2, num_subcores=16
    ),
    scratch_types=[
        pltpu.VMEM((_HE,), jnp.float32),
        pltpu.VMEM((_HE,), jnp.float32),
        pltpu.VMEM((_KPT * 16,), jnp.float32),
        pltpu.VMEM((_OE,), jnp.float32),
    ],
)(_sc_kers_body)


def _tc_reduce_body(kers_ref, day_ref, scal_ref, ll_ref, l1_ref, l2_ref):
    lam0 = scal_ref[0]
    bb = scal_ref[1]
    aa = scal_ref[2]
    kers = kers_ref[...]
    lams1 = jnp.sum(jnp.log(kers + (lam0 + _EPS)))
    day = day_ref[...]
    rem = jnp.clip(jnp.float32(_T) - day, 0.0, jnp.float32(_KPT))
    edo = jnp.sum(aa * (1.0 - jnp.exp(-bb * rem)))
    lams2 = lam0 * (_AREA * _T) + edo
    l1_ref[0, 0] = lams1
    l2_ref[0, 0] = lams2
    ll_ref[0, 0] = lams1 - lams2


_tc_reduce = pl.pallas_call(
    _tc_reduce_body,
    out_shape=[
        jax.ShapeDtypeStruct((1, 1), jnp.float32),
        jax.ShapeDtypeStruct((1, 1), jnp.float32),
        jax.ShapeDtypeStruct((1, 1), jnp.float32),
    ],
    in_specs=[
        pl.BlockSpec(memory_space=pltpu.VMEM),
        pl.BlockSpec(memory_space=pltpu.VMEM),
        pl.BlockSpec(memory_space=pltpu.SMEM),
    ],
    out_specs=[
        pl.BlockSpec(memory_space=pltpu.SMEM),
        pl.BlockSpec(memory_space=pltpu.SMEM),
        pl.BlockSpec(memory_space=pltpu.SMEM),
    ],
)


def kernel(obs, Lambda0, C, beta, sigma):
    lam0 = Lambda0[0]
    c = C[0]
    b = beta[0]
    sg = sigma[0]

    day = obs[:, 0]
    scale = 1.0 / (jnp.sqrt(2.0) * sg)
    xs = obs[:, 1] * scale
    ys = obs[:, 2] * scale
    zpad = jnp.zeros((_KPT * _P,), jnp.float32)
    xp = jnp.concatenate([zpad, xs])
    yp = jnp.concatenate([zpad, ys])

    norm = 1.0 / (2.0 * math.pi * sg * sg)
    ks = jnp.arange(1, _KPT + 1, dtype=jnp.float32)
    wk = c * b * jnp.exp(-b * ks) * norm
    wsplat = jnp.repeat(wk, 16)

    kers = _sc_kers(xp, yp, wsplat)

    eb = jnp.exp(-b)
    aa = c * b * eb / (1.0 - eb)
    scal = jnp.stack([lam0, b, aa])
    ll, l1, l2 = _tc_reduce(
        kers.reshape(_N // 128, 128), day.reshape(_N // 128, 128), scal
    )
    return ll[0, 0], l1[0, 0], l2[0, 0]
